# baseline (speedup 1.0000x reference)
import functools

import jax
import jax.numpy as jnp
from jax import lax
from jax.experimental import pallas as pl
from jax.experimental.pallas import tpu as pltpu
from jax.experimental.pallas import tpu_sc as plsc

_CODEBOOK = 1024
_DIM = 64
_ROWS = 16384
_R = 512
_G = _ROWS // _R
_BETA = 0.25

_NC = 2
_NS = 16
_NW = _NC * _NS
_BPW = _ROWS // _NW


def _argmin_body(x_ref, sx_ref, se_ref, emb_ref, idx_ref, loss_ref):
    i = pl.program_id(0)
    x = x_ref[...]
    t = lax.dot_general(x, emb_ref[...], (((1,), (1,)), ((), ())),
                        preferred_element_type=jnp.float32)
    d = (sx_ref[...] - 2.0 * t) + se_ref[...]
    m = jnp.min(d, axis=1, keepdims=True)
    lanes = lax.broadcasted_iota(jnp.int32, d.shape, 1)
    idx_ref[...] = jnp.min(jnp.where(d == m, lanes, _CODEBOOK), axis=1,
                           keepdims=True)

    part = jnp.sum(m).reshape(1, 1)

    @pl.when(i == 0)
    def _():
        loss_ref[...] = jnp.zeros((1, 1), jnp.float32)

    loss_ref[...] += part

    @pl.when(i == _G - 1)
    def _():
        loss_ref[...] = loss_ref[...] * ((1.0 + _BETA) / float(_ROWS * _DIM))


def _tc_argmin(flat, sx, se, e):
    return pl.pallas_call(
        _argmin_body,
        grid=(_G,),
        in_specs=[
            pl.BlockSpec((_R, _DIM), lambda i: (i, 0)),
            pl.BlockSpec((_R, 1), lambda i: (i, 0)),
            pl.BlockSpec((1, _CODEBOOK), lambda i: (0, 0)),
            pl.BlockSpec((_CODEBOOK, _DIM), lambda i: (0, 0)),
        ],
        out_specs=[
            pl.BlockSpec((_R, 1), lambda i: (i, 0)),
            pl.BlockSpec((1, 1), lambda i: (0, 0)),
        ],
        out_shape=[
            jax.ShapeDtypeStruct((_ROWS, 1), jnp.int32),
            jax.ShapeDtypeStruct((1, 1), jnp.float32),
        ],
    )(flat, sx, se, e)


@functools.partial(
    pl.kernel,
    out_type=jax.ShapeDtypeStruct((_ROWS, _DIM), jnp.float32),
    mesh=plsc.VectorSubcoreMesh(core_axis_name="c", subcore_axis_name="s",
                                num_cores=_NC, num_subcores=_NS),
    scratch_types=[
        pltpu.VMEM((_BPW,), jnp.int32),
        pltpu.VMEM((_BPW, _DIM), jnp.float32),
        pltpu.SemaphoreType.DMA,
    ],
    compiler_params=pltpu.CompilerParams(use_tc_tiling_on_sc=False),
)
def _sc_gather(emb_hbm, idx_hbm, out_hbm, idx_v, rows_v, sem):
    wid = lax.axis_index("s") * _NC + lax.axis_index("c")
    base = wid * _BPW
    pltpu.sync_copy(idx_hbm.at[pl.ds(base, _BPW)], idx_v)
    pltpu.async_copy(emb_hbm.at[idx_v], rows_v, sem).wait()
    pltpu.sync_copy(rows_v, out_hbm.at[pl.ds(base, _BPW)])


def kernel(z_e, emb):
    bsz, num_slots, code_dim = z_e.shape
    flat = z_e.reshape(-1, code_dim).astype(jnp.float32)
    e = emb.astype(jnp.float32)
    sx = jnp.sum(flat ** 2, axis=1, keepdims=True)
    se = jnp.sum(e ** 2, axis=1, keepdims=True).T

    idx, loss = _tc_argmin(flat, sx, se, e)
    idx_flat = idx.reshape(_ROWS)
    zq = _sc_gather(e, idx_flat)

    return (zq.reshape(bsz, num_slots, code_dim),
            idx.reshape(bsz, num_slots),
            loss[0, 0])


# speedup vs baseline: 1.0627x; 1.0627x over previous
import functools

import jax
import jax.numpy as jnp
from jax import lax
from jax.experimental import pallas as pl
from jax.experimental.pallas import tpu as pltpu
from jax.experimental.pallas import tpu_sc as plsc

_CODEBOOK = 1024
_DIM = 64
_ROWS = 16384
_R = 512
_G = _ROWS // _R
_HALVES = 1024 // _R
_BETA = 0.25

_NC = 2
_NS = 16
_NW = _NC * _NS
_BPW = _ROWS // _NW


def _argmin_body(x_ref, emb_ref, se_ref, idx_ref, loss_ref):
    i = pl.program_id(0)
    x = x_ref[...].reshape(_R, _DIM)
    emb = emb_ref[...]
    sx = jnp.sum(x * x, axis=1, keepdims=True)
    t = lax.dot_general(x, emb, (((1,), (1,)), ((), ())),
                        preferred_element_type=jnp.float32)
    d = (sx - 2.0 * t) + se_ref[...]
    m = jnp.min(d, axis=1, keepdims=True)
    lanes = lax.broadcasted_iota(jnp.int32, d.shape, 1)
    idx_ref[...] = jnp.min(jnp.where(d == m, lanes, _CODEBOOK), axis=1,
                           keepdims=True)

    part = jnp.sum(m).reshape(1, 1)

    @pl.when(i == 0)
    def _():
        loss_ref[...] = jnp.zeros((1, 1), jnp.float32)

    loss_ref[...] += part

    @pl.when(i == _G - 1)
    def _():
        loss_ref[...] = loss_ref[...] * ((1.0 + _BETA) / float(_ROWS * _DIM))


def _tc_argmin(z_e3, se, e):
    return pl.pallas_call(
        _argmin_body,
        grid=(_G,),
        in_specs=[
            pl.BlockSpec((1, _R, _DIM),
                         lambda i: (i // _HALVES, i % _HALVES, 0)),
            pl.BlockSpec((_CODEBOOK, _DIM), lambda i: (0, 0)),
            pl.BlockSpec((1, _CODEBOOK), lambda i: (0, 0)),
        ],
        out_specs=[
            pl.BlockSpec((_R, 1), lambda i: (i, 0)),
            pl.BlockSpec((1, 1), lambda i: (0, 0)),
        ],
        out_shape=[
            jax.ShapeDtypeStruct((_ROWS, 1), jnp.int32),
            jax.ShapeDtypeStruct((1, 1), jnp.float32),
        ],
    )(z_e3, e, se)


@functools.partial(
    pl.kernel,
    out_type=jax.ShapeDtypeStruct((_ROWS, _DIM), jnp.float32),
    mesh=plsc.VectorSubcoreMesh(core_axis_name="c", subcore_axis_name="s",
                                num_cores=_NC, num_subcores=_NS),
    scratch_types=[
        pltpu.VMEM((_BPW,), jnp.int32),
        pltpu.VMEM((_BPW, _DIM), jnp.float32),
        pltpu.SemaphoreType.DMA,
    ],
    compiler_params=pltpu.CompilerParams(use_tc_tiling_on_sc=False),
)
def _sc_gather(emb_hbm, idx_hbm, out_hbm, idx_v, rows_v, sem):
    wid = lax.axis_index("s") * _NC + lax.axis_index("c")
    base = wid * _BPW
    pltpu.sync_copy(idx_hbm.at[pl.ds(base, _BPW)], idx_v)
    pltpu.async_copy(emb_hbm.at[idx_v], rows_v, sem).wait()
    pltpu.sync_copy(rows_v, out_hbm.at[pl.ds(base, _BPW)])


def kernel(z_e, emb):
    bsz, num_slots, code_dim = z_e.shape
    z_e3 = z_e.astype(jnp.float32)
    e = emb.astype(jnp.float32)
    se = jnp.sum(e ** 2, axis=1, keepdims=True).T

    idx, loss = _tc_argmin(z_e3, se, e)
    idx_flat = idx.reshape(_ROWS)
    zq = _sc_gather(e, idx_flat)

    return (zq.reshape(bsz, num_slots, code_dim),
            idx.reshape(bsz, num_slots),
            loss[0, 0])
